# Initial kernel scaffold; baseline (speedup 1.0000x reference)
import jax
import jax.numpy as jnp
from jax.experimental import pallas as pl


def kernel(edge_index, node_feats, W0, b0, W1, b1, W2, b2, Wsk, bsk):
    raise NotImplementedError("write your pallas kernel here")


# speedup vs baseline: 4.9530x; 4.9530x over previous
import jax
import jax.numpy as jnp
from jax import lax
from jax.experimental import pallas as pl
from jax.experimental.pallas import tpu as pltpu
from jax.experimental.pallas import tpu_sc as plsc

N = 10000
E = 320000
D = 128
D_OUT = 256

NC = 2
NS = 16
L = 16
NW = NC * NS
EP = E // NW
K = 80
NCH = EP // K
NB = 2
SR = 624
TAIL = N - NS * SR
ZB = 48
DEGW = 16

_MESH = dict(core_axis_name="c", subcore_axis_name="s", num_cores=NC,
             num_subcores=NS)


def _agg_body(xs_hbm, edge_hbm, out_hbm, sidx_v, didx_v, rows_v, zz_v, acc,
              gsems):
    c = lax.axis_index("c")
    s = lax.axis_index("s")
    wid = s * NC + c

    pltpu.sync_copy(edge_hbm.at[0, wid], sidx_v)
    pltpu.sync_copy(edge_hbm.at[1, wid], didx_v)

    def fill_zz(i, carry):
        for t in range(D // L):
            zz_v[i, pl.ds(t * L, L)] = jnp.zeros((L,), jnp.float32)
        return carry

    lax.fori_loop(0, ZB, fill_zz, 0)
    for t in range(SR // ZB):
        pltpu.sync_copy(zz_v, acc.at[pl.ds(s * SR + t * ZB, ZB)])

    @pl.when(s == NS - 1)
    def _():
        pltpu.sync_copy(zz_v.at[pl.ds(0, TAIL)], acc.at[pl.ds(NS * SR, TAIL)])

    plsc.subcore_barrier()

    def scatter_chunk(j, b):
        for u in range(K // L):
            idxv = didx_v[pl.ds(j * K + u * L, L)]
            pltpu.sync_copy(rows_v.at[b, pl.ds(u * L, L)], acc.at[idxv],
                            add=True)

    def group(g, carry):
        cps = []
        for b in range(NB):
            j = g * NB + b
            cps.append(pltpu.async_copy(
                xs_hbm.at[sidx_v.at[pl.ds(j * K, K)]], rows_v.at[b],
                gsems.at[b]))
        for b in range(NB):
            cps[b].wait()
            scatter_chunk(g * NB + b, b)
        return carry

    lax.fori_loop(0, NCH // NB, group, 0)
    cp = pltpu.async_copy(xs_hbm.at[sidx_v.at[pl.ds((NCH - 1) * K, K)]],
                          rows_v.at[0], gsems.at[0])
    cp.wait()
    scatter_chunk(NCH - 1, 0)

    plsc.subcore_barrier()

    pltpu.sync_copy(acc.at[pl.ds(s * SR, SR)],
                    out_hbm.at[c, pl.ds(s * SR, SR)])

    @pl.when(s == NS - 1)
    def _():
        pltpu.sync_copy(acc.at[pl.ds(NS * SR, TAIL)],
                        out_hbm.at[c, pl.ds(NS * SR, TAIL)])


_agg_call = pl.kernel(
    _agg_body,
    out_type=jax.ShapeDtypeStruct((NC, N, D), jnp.float32),
    mesh=plsc.VectorSubcoreMesh(**_MESH),
    scratch_types=[
        pltpu.VMEM((EP,), jnp.int32),
        pltpu.VMEM((EP,), jnp.int32),
        pltpu.VMEM((NB, K, D), jnp.float32),
        pltpu.VMEM((ZB, D), jnp.float32),
        pltpu.VMEM_SHARED((N, D), jnp.float32),
        pltpu.SemaphoreType.DMA((NB,)),
    ],
)


BN = 2000
G = N // BN


def _norms(degp, kind):
    d = degp[kind, 0, :, 0:1] + degp[kind, 1, :, 0:1]
    return jnp.where(d > 0, lax.rsqrt(d), 0.0)


def _prep_body(po_ref, pi_ref, x_ref, xs_ref, degp_ref):
    po16 = po_ref[...][:, :, 0:DEGW]
    pi16 = pi_ref[...][:, :, 0:DEGW]
    degp_ref[0] = po16
    degp_ref[1] = pi16
    d = po16[0, :, 0:1] + po16[1, :, 0:1]
    ns = jnp.where(d > 0, lax.rsqrt(d), 0.0)
    xs_ref[...] = x_ref[...] * ns


_prep_call = pl.pallas_call(
    _prep_body,
    grid=(G,),
    in_specs=[
        pl.BlockSpec((NC, BN, D), lambda i: (0, i, 0)),
        pl.BlockSpec((NC, BN, D), lambda i: (0, i, 0)),
        pl.BlockSpec((BN, D), lambda i: (i, 0)),
    ],
    out_specs=[
        pl.BlockSpec((BN, D), lambda i: (i, 0)),
        pl.BlockSpec((2, NC, BN, DEGW), lambda i: (0, 0, i, 0)),
    ],
    out_shape=[
        jax.ShapeDtypeStruct((N, D), jnp.float32),
        jax.ShapeDtypeStruct((2, NC, N, DEGW), jnp.float32),
    ],
)


def _l0_body(p_ref, degp_ref, w_ref, b_ref, h0_ref, x1s_ref):
    degp = degp_ref[...]
    agg = (p_ref[0] + p_ref[1]) * _norms(degp, 1)
    h = jnp.dot(agg, w_ref[...], preferred_element_type=jnp.float32) + b_ref[...]
    h0 = jnp.maximum(h, 0.0)
    h0_ref[...] = h0
    x1s_ref[...] = h0 * _norms(degp, 0)


_l0_call = pl.pallas_call(
    _l0_body,
    grid=(G,),
    in_specs=[
        pl.BlockSpec((NC, BN, D), lambda i: (0, i, 0)),
        pl.BlockSpec((2, NC, BN, DEGW), lambda i: (0, 0, i, 0)),
        pl.BlockSpec((D, D), lambda i: (0, 0)),
        pl.BlockSpec((1, D), lambda i: (0, 0)),
    ],
    out_specs=[
        pl.BlockSpec((BN, D), lambda i: (i, 0)),
        pl.BlockSpec((BN, D), lambda i: (i, 0)),
    ],
    out_shape=[
        jax.ShapeDtypeStruct((N, D), jnp.float32),
        jax.ShapeDtypeStruct((N, D), jnp.float32),
    ],
)


def _l1_body(p_ref, degp_ref, h0_ref, w_ref, b_ref, h1_ref, x2s_ref):
    degp = degp_ref[...]
    agg = (p_ref[0] + p_ref[1]) * _norms(degp, 1)
    gcn = jnp.dot(agg, w_ref[...], preferred_element_type=jnp.float32) + b_ref[...]
    h1 = jnp.maximum(0.6 * h0_ref[...] + 0.4 * gcn, 0.0)
    h1_ref[...] = h1
    x2s_ref[...] = h1 * _norms(degp, 0)


_l1_call = pl.pallas_call(
    _l1_body,
    grid=(G,),
    in_specs=[
        pl.BlockSpec((NC, BN, D), lambda i: (0, i, 0)),
        pl.BlockSpec((2, NC, BN, DEGW), lambda i: (0, 0, i, 0)),
        pl.BlockSpec((BN, D), lambda i: (i, 0)),
        pl.BlockSpec((D, D), lambda i: (0, 0)),
        pl.BlockSpec((1, D), lambda i: (0, 0)),
    ],
    out_specs=[
        pl.BlockSpec((BN, D), lambda i: (i, 0)),
        pl.BlockSpec((BN, D), lambda i: (i, 0)),
    ],
    out_shape=[
        jax.ShapeDtypeStruct((N, D), jnp.float32),
        jax.ShapeDtypeStruct((N, D), jnp.float32),
    ],
)


def _l2_body(p_ref, degp_ref, h1_ref, w_ref, b_ref, wsk_ref, bsk_ref, out_ref):
    degp = degp_ref[...]
    agg = (p_ref[0] + p_ref[1]) * _norms(degp, 1)
    gcn = jnp.dot(agg, w_ref[...], preferred_element_type=jnp.float32) + b_ref[...]
    skip = jnp.dot(h1_ref[...], wsk_ref[...],
                   preferred_element_type=jnp.float32) + bsk_ref[...]
    out_ref[...] = 0.6 * skip + 0.4 * gcn


_l2_call = pl.pallas_call(
    _l2_body,
    grid=(G,),
    in_specs=[
        pl.BlockSpec((NC, BN, D), lambda i: (0, i, 0)),
        pl.BlockSpec((2, NC, BN, DEGW), lambda i: (0, 0, i, 0)),
        pl.BlockSpec((BN, D), lambda i: (i, 0)),
        pl.BlockSpec((D, D_OUT), lambda i: (0, 0)),
        pl.BlockSpec((1, D_OUT), lambda i: (0, 0)),
        pl.BlockSpec((D, D_OUT), lambda i: (0, 0)),
        pl.BlockSpec((1, D_OUT), lambda i: (0, 0)),
    ],
    out_specs=pl.BlockSpec((BN, D_OUT), lambda i: (i, 0)),
    out_shape=jax.ShapeDtypeStruct((N, D_OUT), jnp.float32),
)


def kernel(edge_index, node_feats, W0, b0, W1, b1, W2, b2, Wsk, bsk):
    ei = edge_index.astype(jnp.int32)
    edge3 = ei.reshape(2, NW, EP)
    edge3r = edge3[::-1]

    ones_feat = jnp.ones((N, D), jnp.float32)
    po = _agg_call(ones_feat, edge3r)
    pi = _agg_call(ones_feat, edge3)
    x0s, degp = _prep_call(po, pi, node_feats)
    p0 = _agg_call(x0s, edge3)
    h0, x1s = _l0_call(p0, degp, W0, b0.reshape(1, D))
    p1 = _agg_call(x1s, edge3)
    h1, x2s = _l1_call(p1, degp, h0, W1, b1.reshape(1, D))
    p2 = _agg_call(x2s, edge3)
    out = _l2_call(p2, degp, h1, W2, b2.reshape(1, D_OUT),
                   Wsk, bsk.reshape(1, D_OUT))
    return out[None]
